# baseline (speedup 1.0000x reference)
import functools

import jax
import jax.numpy as jnp
from jax import lax
from jax.experimental import pallas as pl
from jax.experimental.pallas import tpu as pltpu
from jax.experimental.pallas import tpu_sc as plsc

N_USERS = 1000000
N_ITEMS = 1000000
EMBED_DIM = 16
N_CLIMATE = 4
BATCH = 16384

NUM_CORES = 2
NUM_SUBCORES = 16
LANES = 16
NW = NUM_CORES * NUM_SUBCORES
B_PER_W = BATCH // NW
CHUNKS = B_PER_W // LANES


@functools.partial(
    pl.kernel,
    out_type=jax.ShapeDtypeStruct((BATCH,), jnp.float32),
    mesh=plsc.VectorSubcoreMesh(core_axis_name="c", subcore_axis_name="s"),
    compiler_params=pltpu.CompilerParams(
        needs_layout_passes=False, use_tc_tiling_on_sc=False),
    scratch_types=[
        pltpu.VMEM((B_PER_W // 128, 128), jnp.int32),
        pltpu.VMEM((B_PER_W // 128, 128), jnp.int32),
        pltpu.VMEM((B_PER_W, EMBED_DIM), jnp.float32),
        pltpu.VMEM((B_PER_W, EMBED_DIM), jnp.float32),
        pltpu.VMEM((B_PER_W,), jnp.float32),
        pltpu.VMEM((B_PER_W,), jnp.float32),
        pltpu.VMEM((B_PER_W, N_CLIMATE), jnp.float32),
        pltpu.VMEM((N_CLIMATE, LANES), jnp.float32),
        pltpu.VMEM((LANES,), jnp.float32),
        pltpu.VMEM((B_PER_W,), jnp.float32),
        pltpu.SemaphoreType.DMA,
    ],
)
def _fm_kernel(uids_hbm, iids_hbm, clim_hbm, uemb_hbm, iemb_hbm,
               ubias_hbm, ibias_hbm, wbc_hbm, cbc_hbm, out_hbm,
               uid_v, iid_v, urows_v, irows_v, ub_v, ib_v, clim_v,
               w_v, const_v, out_v, sem):
    wid = lax.axis_index("s") * NUM_CORES + lax.axis_index("c")
    base = wid * B_PER_W

    pltpu.sync_copy(uids_hbm.at[wid], uid_v)
    pltpu.sync_copy(iids_hbm.at[wid], iid_v)
    pltpu.sync_copy(clim_hbm.at[pl.ds(base, B_PER_W)], clim_v)
    pltpu.sync_copy(wbc_hbm, w_v)
    pltpu.sync_copy(cbc_hbm, const_v)

    copies = []
    for j in range(B_PER_W // 128):
        sl = pl.ds(j * 128, 128)
        copies.append(pltpu.async_copy(uemb_hbm.at[uid_v.at[j]], urows_v.at[sl], sem))
        copies.append(pltpu.async_copy(iemb_hbm.at[iid_v.at[j]], irows_v.at[sl], sem))
        copies.append(pltpu.async_copy(ubias_hbm.at[uid_v.at[j]], ub_v.at[sl], sem))
        copies.append(pltpu.async_copy(ibias_hbm.at[iid_v.at[j]], ib_v.at[sl], sem))
    for cp in copies:
        cp.wait()

    iota = jnp.arange(LANES, dtype=jnp.int32)

    def chunk_body(c, carry):
        rows = c * LANES + iota
        acc = const_v[...]
        for cc in range(N_CLIMATE):
            f = plsc.load_gather(clim_v, [rows, jnp.full((LANES,), cc, jnp.int32)])
            acc = acc + f * w_v[cc]
        for dd in range(EMBED_DIM):
            col = jnp.full((LANES,), dd, jnp.int32)
            u = plsc.load_gather(urows_v, [rows, col])
            v = plsc.load_gather(irows_v, [rows, col])
            acc = acc + u * v
        ub = plsc.load_gather(ub_v, [rows])
        ib = plsc.load_gather(ib_v, [rows])
        out_v[pl.ds(c * LANES, LANES)] = acc + ub + ib
        return carry

    lax.fori_loop(0, CHUNKS, chunk_body, 0)

    pltpu.sync_copy(out_v, out_hbm.at[pl.ds(base, B_PER_W)])


def kernel(user_ids, item_ids, climate_feats, user_emb, item_emb,
           user_bias, item_bias, W_climate, b_climate, global_bias):
    w_bcast = jnp.broadcast_to(W_climate.reshape(N_CLIMATE, 1), (N_CLIMATE, LANES))
    const_bcast = jnp.broadcast_to(b_climate + global_bias, (LANES,))
    out = _fm_kernel(
        user_ids.astype(jnp.int32).reshape(NW, B_PER_W // 128, 128),
        item_ids.astype(jnp.int32).reshape(NW, B_PER_W // 128, 128),
        climate_feats, user_emb, item_emb,
        user_bias.reshape(N_USERS), item_bias.reshape(N_ITEMS),
        w_bcast, const_bcast,
    )
    return out.reshape(BATCH, 1)


# speedup vs baseline: 3.2943x; 3.2943x over previous
import functools

import jax
import jax.numpy as jnp
from jax import lax
from jax.experimental import pallas as pl
from jax.experimental.pallas import tpu as pltpu
from jax.experimental.pallas import tpu_sc as plsc

N_USERS = 1000000
N_ITEMS = 1000000
EMBED_DIM = 16
N_CLIMATE = 4
BATCH = 16384

NUM_CORES = 2
NUM_SUBCORES = 16
LANES = 16
NW = NUM_CORES * NUM_SUBCORES
B_PER_W = BATCH // NW
CHUNKS = B_PER_W // LANES


@functools.partial(
    pl.kernel,
    out_type=jax.ShapeDtypeStruct((BATCH,), jnp.float32),
    mesh=plsc.VectorSubcoreMesh(core_axis_name="c", subcore_axis_name="s"),
    compiler_params=pltpu.CompilerParams(
        needs_layout_passes=False, use_tc_tiling_on_sc=True),
    scratch_types=[
        pltpu.VMEM((B_PER_W,), jnp.int32),
        pltpu.VMEM((B_PER_W,), jnp.int32),
        pltpu.VMEM((LANES, EMBED_DIM, 128), jnp.float32),
        pltpu.VMEM((LANES, EMBED_DIM, 128), jnp.float32),
        pltpu.VMEM((B_PER_W,), jnp.float32),
        pltpu.VMEM((B_PER_W,), jnp.float32),
        pltpu.VMEM((N_CLIMATE * B_PER_W,), jnp.float32),
        pltpu.VMEM((5 * LANES,), jnp.float32),
        pltpu.VMEM((LANES, LANES), jnp.float32),
        pltpu.VMEM((B_PER_W,), jnp.float32),
        pltpu.SemaphoreType.DMA,
        pltpu.SemaphoreType.DMA,
    ],
)
def _fm_kernel(uids_hbm, iids_hbm, clim_hbm, uembT_hbm, iembT_hbm,
               ubias_hbm, ibias_hbm, params_hbm, out_hbm,
               uid_v, iid_v, ublk_v, iblk_v, ub_v, ib_v, clim_v,
               params_v, pmat_v, out_v, sem, bsem):
    wid = lax.axis_index("s") * NUM_CORES + lax.axis_index("c")
    base = wid * B_PER_W

    pltpu.sync_copy(uids_hbm.at[pl.ds(base, B_PER_W)], uid_v)
    pltpu.sync_copy(iids_hbm.at[pl.ds(base, B_PER_W)], iid_v)
    pltpu.sync_copy(clim_hbm.at[pl.ds(base * N_CLIMATE, B_PER_W * N_CLIMATE)],
                    clim_v)
    pltpu.sync_copy(params_hbm, params_v)

    copies = []
    for j in range(B_PER_W // 128):
        sl = pl.ds(j * 128, 128)
        copies.append(pltpu.async_copy(ubias_hbm.at[uid_v.at[sl]], ub_v.at[sl], sem))
        copies.append(pltpu.async_copy(ibias_hbm.at[iid_v.at[sl]], ib_v.at[sl], sem))
    for cp in copies:
        cp.wait()

    iota = jnp.arange(LANES, dtype=jnp.int32)

    def chunk_body(c, carry):
        rows = c * LANES + iota
        uids_vec = uid_v[pl.ds(c * LANES, LANES)]
        iids_vec = iid_v[pl.ds(c * LANES, LANES)]

        fetches = []
        for e in range(LANES):
            ucol0 = pl.multiple_of((uids_vec[e] // 128) * 128, 128)
            icol0 = pl.multiple_of((iids_vec[e] // 128) * 128, 128)
            fetches.append(pltpu.async_copy(
                uembT_hbm.at[:, pl.ds(ucol0, 128)], ublk_v.at[e], bsem))
            fetches.append(pltpu.async_copy(
                iembT_hbm.at[:, pl.ds(icol0, 128)], iblk_v.at[e], bsem))
        for cp in fetches:
            cp.wait()

        for e in range(LANES):
            ulane = jnp.full((LANES,), uids_vec[e] % 128, jnp.int32)
            ilane = jnp.full((LANES,), iids_vec[e] % 128, jnp.int32)
            u = plsc.load_gather(ublk_v.at[e], [iota, ulane])
            v = plsc.load_gather(iblk_v.at[e], [iota, ilane])
            plsc.store_scatter(pmat_v, [iota, jnp.full((LANES,), e, jnp.int32)],
                               u * v)

        dot = pmat_v[0]
        for r in range(1, LANES):
            dot = dot + pmat_v[r]

        acc = dot + params_v[pl.ds(4 * LANES, LANES)]
        for cc in range(N_CLIMATE):
            f = plsc.load_gather(clim_v, [rows * N_CLIMATE + cc])
            acc = acc + f * params_v[pl.ds(cc * LANES, LANES)]
        ub = plsc.load_gather(ub_v, [rows])
        ib = plsc.load_gather(ib_v, [rows])
        out_v[pl.ds(c * LANES, LANES)] = acc + ub + ib
        return carry

    lax.fori_loop(0, CHUNKS, chunk_body, 0)

    pltpu.sync_copy(out_v, out_hbm.at[pl.ds(base, B_PER_W)])


def kernel(user_ids, item_ids, climate_feats, user_emb, item_emb,
           user_bias, item_bias, W_climate, b_climate, global_bias):
    w_bcast = jnp.broadcast_to(W_climate.reshape(N_CLIMATE, 1), (N_CLIMATE, LANES))
    const_bcast = jnp.broadcast_to(b_climate + global_bias, (1, LANES))
    params = jnp.concatenate([w_bcast, const_bcast], axis=0).reshape(5 * LANES)
    out = _fm_kernel(
        user_ids.astype(jnp.int32), item_ids.astype(jnp.int32),
        climate_feats.reshape(BATCH * N_CLIMATE),
        user_emb.T, item_emb.T,
        user_bias.reshape(N_USERS), item_bias.reshape(N_ITEMS),
        params,
    )
    return out.reshape(BATCH, 1)


# speedup vs baseline: 4.8569x; 1.4743x over previous
import functools

import jax
import jax.numpy as jnp
from jax import lax
from jax.experimental import pallas as pl
from jax.experimental.pallas import tpu as pltpu
from jax.experimental.pallas import tpu_sc as plsc

N_USERS = 1000000
N_ITEMS = 1000000
EMBED_DIM = 16
N_CLIMATE = 4
BATCH = 16384

NUM_CORES = 2
NUM_SUBCORES = 16
LANES = 16
NW = NUM_CORES * NUM_SUBCORES
B_PER_W = BATCH // NW
CHUNKS = B_PER_W // LANES


@functools.partial(
    pl.kernel,
    out_type=jax.ShapeDtypeStruct((BATCH,), jnp.float32),
    mesh=plsc.VectorSubcoreMesh(core_axis_name="c", subcore_axis_name="s"),
    compiler_params=pltpu.CompilerParams(
        needs_layout_passes=False, use_tc_tiling_on_sc=False),
    scratch_types=[
        pltpu.VMEM((B_PER_W,), jnp.int32),
        pltpu.VMEM((B_PER_W,), jnp.int32),
        pltpu.VMEM((B_PER_W,), jnp.float32),
        pltpu.VMEM((B_PER_W,), jnp.float32),
        pltpu.VMEM((B_PER_W,), jnp.float32),
        pltpu.SemaphoreType.DMA,
    ],
)
def _bias_kernel(uids_hbm, iids_hbm, ubiasT_hbm, ibiasT_hbm, out_hbm,
                 uid_v, iid_v, ub_v, ib_v, out_v, sem):
    wid = lax.axis_index("s") * NUM_CORES + lax.axis_index("c")
    base = wid * B_PER_W

    pltpu.sync_copy(uids_hbm.at[pl.ds(base, B_PER_W)], uid_v)
    pltpu.sync_copy(iids_hbm.at[pl.ds(base, B_PER_W)], iid_v)

    ub1 = ubiasT_hbm.at[0]
    ib1 = ibiasT_hbm.at[0]
    copies = []
    for j in range(B_PER_W // 128):
        sl = pl.ds(j * 128, 128)
        copies.append(pltpu.async_copy(ub1.at[uid_v.at[sl]], ub_v.at[sl], sem))
        copies.append(pltpu.async_copy(ib1.at[iid_v.at[sl]], ib_v.at[sl], sem))
    for cp in copies:
        cp.wait()

    iota = jnp.arange(LANES, dtype=jnp.int32)

    def chunk_body(c, carry):
        rows = c * LANES + iota
        ub = plsc.load_gather(ub_v, [rows])
        ib = plsc.load_gather(ib_v, [rows])
        out_v[pl.ds(c * LANES, LANES)] = ub + ib
        return carry

    lax.fori_loop(0, CHUNKS, chunk_body, 0)
    pltpu.sync_copy(out_v, out_hbm.at[pl.ds(base, B_PER_W)])


@functools.partial(
    pl.kernel,
    out_type=jax.ShapeDtypeStruct((BATCH,), jnp.float32),
    mesh=plsc.VectorSubcoreMesh(core_axis_name="c", subcore_axis_name="s"),
    compiler_params=pltpu.CompilerParams(
        needs_layout_passes=False, use_tc_tiling_on_sc=True),
    scratch_types=[
        pltpu.VMEM((B_PER_W,), jnp.int32),
        pltpu.VMEM((B_PER_W,), jnp.int32),
        pltpu.VMEM((LANES, EMBED_DIM, 128), jnp.float32),
        pltpu.VMEM((LANES, EMBED_DIM, 128), jnp.float32),
        pltpu.VMEM((N_CLIMATE * B_PER_W,), jnp.float32),
        pltpu.VMEM((5 * LANES,), jnp.float32),
        pltpu.VMEM((LANES, LANES), jnp.float32),
        pltpu.VMEM((B_PER_W,), jnp.float32),
        pltpu.SemaphoreType.DMA,
    ],
)
def _fm_kernel(uids_hbm, iids_hbm, clim_hbm, uembT_hbm, iembT_hbm,
               params_hbm, out_hbm,
               uid_v, iid_v, ublk_v, iblk_v, clim_v,
               params_v, pmat_v, out_v, bsem):
    wid = lax.axis_index("s") * NUM_CORES + lax.axis_index("c")
    base = wid * B_PER_W

    pltpu.sync_copy(uids_hbm.at[pl.ds(base, B_PER_W)], uid_v)
    pltpu.sync_copy(iids_hbm.at[pl.ds(base, B_PER_W)], iid_v)
    pltpu.sync_copy(clim_hbm.at[pl.ds(base * N_CLIMATE, B_PER_W * N_CLIMATE)],
                    clim_v)
    pltpu.sync_copy(params_hbm, params_v)

    iota = jnp.arange(LANES, dtype=jnp.int32)

    def chunk_body(c, carry):
        rows = c * LANES + iota
        uids_vec = uid_v[pl.ds(c * LANES, LANES)]
        iids_vec = iid_v[pl.ds(c * LANES, LANES)]

        fetches = []
        for e in range(LANES):
            ucol0 = pl.multiple_of((uids_vec[e] // 128) * 128, 128)
            icol0 = pl.multiple_of((iids_vec[e] // 128) * 128, 128)
            fetches.append(pltpu.async_copy(
                uembT_hbm.at[:, pl.ds(ucol0, 128)], ublk_v.at[e], bsem))
            fetches.append(pltpu.async_copy(
                iembT_hbm.at[:, pl.ds(icol0, 128)], iblk_v.at[e], bsem))
        for cp in fetches:
            cp.wait()

        for e in range(LANES):
            ulane = jnp.full((LANES,), uids_vec[e] % 128, jnp.int32)
            ilane = jnp.full((LANES,), iids_vec[e] % 128, jnp.int32)
            u = plsc.load_gather(ublk_v.at[e], [iota, ulane])
            v = plsc.load_gather(iblk_v.at[e], [iota, ilane])
            plsc.store_scatter(pmat_v, [iota, jnp.full((LANES,), e, jnp.int32)],
                               u * v)

        dot = pmat_v[0]
        for r in range(1, LANES):
            dot = dot + pmat_v[r]

        acc = dot + params_v[pl.ds(4 * LANES, LANES)]
        for cc in range(N_CLIMATE):
            f = plsc.load_gather(clim_v, [rows * N_CLIMATE + cc])
            acc = acc + f * params_v[pl.ds(cc * LANES, LANES)]
        out_v[pl.ds(c * LANES, LANES)] = acc
        return carry

    lax.fori_loop(0, CHUNKS, chunk_body, 0)

    pltpu.sync_copy(out_v, out_hbm.at[pl.ds(base, B_PER_W)])


def kernel(user_ids, item_ids, climate_feats, user_emb, item_emb,
           user_bias, item_bias, W_climate, b_climate, global_bias):
    w_bcast = jnp.broadcast_to(W_climate.reshape(N_CLIMATE, 1), (N_CLIMATE, LANES))
    const_bcast = jnp.broadcast_to(b_climate + global_bias, (1, LANES))
    params = jnp.concatenate([w_bcast, const_bcast], axis=0).reshape(5 * LANES)
    uids32 = user_ids.astype(jnp.int32)
    iids32 = item_ids.astype(jnp.int32)
    fm = _fm_kernel(
        uids32, iids32,
        climate_feats.reshape(BATCH * N_CLIMATE),
        user_emb.T, item_emb.T,
        params,
    )
    bias = _bias_kernel(uids32, iids32, user_bias.T, item_bias.T)
    return (fm + bias).reshape(BATCH, 1)
